# baseline (speedup 1.0000x reference)
import functools

import jax
import jax.numpy as jnp
from jax import lax
from jax.experimental import pallas as pl
from jax.experimental.pallas import tpu as pltpu
from jax.experimental.pallas import tpu_sc as plsc

FG = 128
BG = 128
_B, _C, _K = 64, 512, 1024
_P = 1024
_I32_MIN = -2147483648
_M31 = 2147483647

_NW = 32
_RPW = _B * FG // _NW
_CHUNK = 16
_NCHUNK = _RPW // _CHUNK
_BPW = _B // _NW


def _sim_kernel(att_ref, aud_ref, sim_ref):
    a = att_ref[0]
    aud = aud_ref[0]
    dot = jnp.dot(aud, a, preferred_element_type=jnp.float32)
    nsq = jnp.sum(a * a, axis=0, keepdims=True)
    sim_ref[0] = dot / jnp.maximum(jnp.sqrt(nsq), 1e-12)


def _key(x_i32):
    return jnp.where(x_i32 < 0, x_i32 ^ jnp.int32(_M31), x_i32)


def _select_kernel(sim_ref, dis_ref, rank_ref):
    sim = sim_ref[...]
    ka = _key(lax.bitcast_convert_type(sim, jnp.int32))

    def body(j, P):
        bit = 31 - j
        phi, plo = P
        step = jnp.int32(1) << bit
        chi = phi + step
        clo = plo + step
        cnt_hi = jnp.sum((ka >= chi).astype(jnp.int32), axis=1, keepdims=True)
        cnt_lo = jnp.sum((ka >= clo).astype(jnp.int32), axis=1, keepdims=True)
        phi = jnp.where(cnt_hi >= FG, chi, phi)
        plo = jnp.where(cnt_lo >= _K - BG + 1, clo, plo)
        return (phi, plo)

    p0 = jnp.full((_B, 1), _I32_MIN, jnp.int32)
    phi, plo = lax.fori_loop(0, 32, body, (p0, p0))

    thi_f = lax.bitcast_convert_type(_key(phi), jnp.float32)
    tlo_f = lax.bitcast_convert_type(_key(plo), jnp.float32)

    gt = ka > phi
    cnt_gt = jnp.sum(gt.astype(jnp.float32), axis=1, keepdims=True)
    sum_gt = jnp.sum(jnp.where(gt, sim, 0.0), axis=1, keepdims=True)
    pos = (sum_gt + thi_f * (FG - cnt_gt)) * (1.0 / FG)

    lt = ka < plo
    cnt_lt = jnp.sum(lt.astype(jnp.float32), axis=1, keepdims=True)
    sum_lt = jnp.sum(jnp.where(lt, sim, 0.0), axis=1, keepdims=True)
    hn = (sum_lt + tlo_f * (BG - cnt_lt)) * (1.0 / BG)

    m = jnp.maximum(pos, hn)
    logz = m + jnp.log(jnp.exp(pos - m) + jnp.exp(hn - m))
    dis = jnp.mean(logz - pos)
    dis_ref[...] = jnp.full((8, 128), dis, jnp.float32)

    eq = (ka == phi)
    row = lax.broadcasted_iota(jnp.int32, (_K, _K), 0)
    col = lax.broadcasted_iota(jnp.int32, (_K, _K), 1)
    tri = (row <= col).astype(jnp.float32)
    stacked = jnp.concatenate(
        [gt.astype(jnp.float32), eq.astype(jnp.float32)], axis=0)
    cums = lax.dot_general(
        stacked, tri, (((1,), (0,)), ((), ())),
        precision=lax.Precision.HIGHEST)
    cum_gt = cums[:_B]
    cum_eq = cums[_B:]
    r = FG - cnt_gt
    sel_eq = eq & (cum_eq <= r + 0.5)
    sel = gt | sel_eq
    cum_sel = cum_gt + jnp.minimum(cum_eq, r)
    rank_ref[...] = jnp.where(sel, cum_sel, 0.0)


def _compact_kernel(rank_ref, idx_ref):
    b = pl.program_id(0)
    rank = rank_ref[0]
    target = lax.broadcasted_iota(jnp.int32, (FG, _K), 0).astype(
        jnp.float32) + 1.0
    onehot = (jnp.broadcast_to(rank, (FG, _K)) == target).astype(jnp.float32)
    kvec = lax.broadcasted_iota(jnp.int32, (_K, 1), 0).astype(jnp.float32)
    kidx = lax.dot_general(onehot, kvec, (((1,), (0,)), ((), ())),
                           precision=lax.Precision.HIGHEST)
    del b
    idx_ref[0] = kidx.astype(jnp.int32)


def _make_gather():
    mesh = plsc.VectorSubcoreMesh(core_axis_name="c", subcore_axis_name="s")

    grp = 8
    ngrp = FG // grp

    @functools.partial(
        pl.kernel, mesh=mesh,
        out_type=jax.ShapeDtypeStruct((_B, 32, 32), jnp.float32),
        scratch_types=[
            pltpu.VMEM((_BPW, FG), jnp.int32),
            pltpu.VMEM((2, grp, 32, 32), jnp.float32),
            pltpu.VMEM((_BPW, 32, 32), jnp.float32),
            pltpu.SemaphoreType.DMA((2,)),
        ],
    )
    def gather(idx_hbm, hm_hbm, out_hbm, idx_s, buf, acc, sems):
        cid = lax.axis_index("c")
        sid = lax.axis_index("s")
        wid = sid * 2 + cid
        b0 = wid * _BPW
        pltpu.sync_copy(idx_hbm.at[pl.ds(b0, _BPW)], idx_s)
        for g in range(_BPW * _P // 16):
            p = g % (_P // 16)
            acc[g // (_P // 16), p // 2, pl.ds((p % 2) * 16, 16)] = (
                jnp.zeros((16,), jnp.float32))

        for s in range(_BPW):

            def hg_body(hg, carry, s=s):
                kv = idx_s[s, pl.ds(hg * 16, 16)]
                for par in range(2):
                    for e in range(grp):
                        pltpu.async_copy(
                            hm_hbm.at[b0 + s, kv[par * grp + e]],
                            buf.at[par, e], sems.at[par])
                for par in range(2):
                    pltpu.make_async_copy(hm_hbm.at[0, pl.ds(0, grp)],
                                          buf.at[par], sems.at[par]).wait()

                    def row_body(rr, c, s=s, par=par):
                        for g in range(_P // 16):
                            acc[s, g // 2, pl.ds((g % 2) * 16, 16)] += (
                                buf[par, rr, g // 2, pl.ds((g % 2) * 16, 16)])
                        return c

                    lax.fori_loop(0, grp, row_body, 0)
                return carry

            lax.fori_loop(0, ngrp // 2, hg_body, 0)
        pltpu.sync_copy(acc, out_hbm.at[pl.ds(b0, _BPW)])

    return gather


def _js_kernel(cs_ref, av_ref, out_ref):
    comb = cs_ref[...] * (1.0 / FG)
    cmax = jnp.max(comb, axis=(1, 2), keepdims=True)
    ce = jnp.exp(comb - cmax)
    att = ce / jnp.sum(ce, axis=(1, 2), keepdims=True)

    av = av_ref[...]
    vmax = jnp.max(av, axis=(1, 2), keepdims=True)
    ve = jnp.exp(av - vmax)
    avd = ve / jnp.sum(ve, axis=(1, 2), keepdims=True)

    lm = jnp.log((att + avd) * 0.5)
    tot = jnp.sum(att * (jnp.log(att) - lm) + avd * (jnp.log(avd) - lm))
    out_ref[...] = jnp.full((8, 128), tot, jnp.float32)


def kernel(att_feat, aud_feat, att_heatmaps, av_heatmaps):
    B, C, K = att_feat.shape
    P = att_heatmaps.shape[2] * att_heatmaps.shape[3]

    sim = pl.pallas_call(
        _sim_kernel,
        grid=(B,),
        in_specs=[
            pl.BlockSpec((1, C, K), lambda b: (b, 0, 0)),
            pl.BlockSpec((1, 1, C), lambda b: (b, 0, 0)),
        ],
        out_specs=pl.BlockSpec((1, 1, K), lambda b: (b, 0, 0)),
        out_shape=jax.ShapeDtypeStruct((B, 1, K), jnp.float32),
    )(att_feat, aud_feat.reshape(B, 1, C)).reshape(B, K)

    dis, rank = pl.pallas_call(
        _select_kernel,
        in_specs=[pl.BlockSpec((B, K), lambda: (0, 0))],
        out_specs=[
            pl.BlockSpec((8, 128), lambda: (0, 0)),
            pl.BlockSpec((B, K), lambda: (0, 0)),
        ],
        out_shape=[
            jax.ShapeDtypeStruct((8, 128), jnp.float32),
            jax.ShapeDtypeStruct((B, K), jnp.float32),
        ],
    )(sim)

    idx = pl.pallas_call(
        _compact_kernel,
        grid=(B,),
        in_specs=[pl.BlockSpec((1, 1, K), lambda b: (b, 0, 0))],
        out_specs=pl.BlockSpec((1, FG, 1), lambda b: (b, 0, 0)),
        out_shape=jax.ShapeDtypeStruct((B, FG, 1), jnp.int32),
    )(rank.reshape(B, 1, K))

    comb_sum = _make_gather()(idx.reshape(B, FG), att_heatmaps)

    div = pl.pallas_call(
        _js_kernel,
        in_specs=[
            pl.BlockSpec((B, 32, 32), lambda: (0, 0, 0)),
            pl.BlockSpec((B, 32, 32), lambda: (0, 0, 0)),
        ],
        out_specs=pl.BlockSpec((8, 128), lambda: (0, 0)),
        out_shape=jax.ShapeDtypeStruct((8, 128), jnp.float32),
    )(comb_sum, av_heatmaps.reshape(B, 32, 32))

    dis_loss = dis[0, 0].reshape(())
    div_loss = (div[0, 0] / (2.0 * B)).reshape(())
    return dis_loss, div_loss


# speedup vs baseline: 1.0164x; 1.0164x over previous
import functools

import jax
import jax.numpy as jnp
from jax import lax
from jax.experimental import pallas as pl
from jax.experimental.pallas import tpu as pltpu
from jax.experimental.pallas import tpu_sc as plsc

FG = 128
BG = 128
_B, _C, _K = 64, 512, 1024
_P = 1024
_I32_MIN = -2147483648
_M31 = 2147483647

_NW = 32
_RPW = _B * FG // _NW
_CHUNK = 16
_NCHUNK = _RPW // _CHUNK
_BPW = _B // _NW


def _sim_kernel(att_ref, aud_ref, sim_ref):
    a = att_ref[0]
    aud = aud_ref[0]
    dot = jnp.dot(aud, a, preferred_element_type=jnp.float32)
    nsq = jnp.sum(a * a, axis=0, keepdims=True)
    sim_ref[0] = dot / jnp.maximum(jnp.sqrt(nsq), 1e-12)


def _key(x_i32):
    return jnp.where(x_i32 < 0, x_i32 ^ jnp.int32(_M31), x_i32)


def _select_kernel(sim_ref, dis_ref, rank_ref):
    sim = sim_ref[...]
    ka = _key(lax.bitcast_convert_type(sim, jnp.int32))

    def body(j, P):
        bit = 31 - j
        phi, plo = P
        step = jnp.int32(1) << bit
        chi = phi + step
        clo = plo + step
        cnt_hi = jnp.sum((ka >= chi).astype(jnp.int32), axis=1, keepdims=True)
        cnt_lo = jnp.sum((ka >= clo).astype(jnp.int32), axis=1, keepdims=True)
        phi = jnp.where(cnt_hi >= FG, chi, phi)
        plo = jnp.where(cnt_lo >= _K - BG + 1, clo, plo)
        return (phi, plo)

    p0 = jnp.full((_B, 1), _I32_MIN, jnp.int32)
    phi, plo = lax.fori_loop(0, 32, body, (p0, p0))

    thi_f = lax.bitcast_convert_type(_key(phi), jnp.float32)
    tlo_f = lax.bitcast_convert_type(_key(plo), jnp.float32)

    gt = ka > phi
    cnt_gt = jnp.sum(gt.astype(jnp.float32), axis=1, keepdims=True)
    sum_gt = jnp.sum(jnp.where(gt, sim, 0.0), axis=1, keepdims=True)
    pos = (sum_gt + thi_f * (FG - cnt_gt)) * (1.0 / FG)

    lt = ka < plo
    cnt_lt = jnp.sum(lt.astype(jnp.float32), axis=1, keepdims=True)
    sum_lt = jnp.sum(jnp.where(lt, sim, 0.0), axis=1, keepdims=True)
    hn = (sum_lt + tlo_f * (BG - cnt_lt)) * (1.0 / BG)

    m = jnp.maximum(pos, hn)
    logz = m + jnp.log(jnp.exp(pos - m) + jnp.exp(hn - m))
    dis = jnp.mean(logz - pos)
    dis_ref[...] = jnp.full((8, 128), dis, jnp.float32)

    eq = (ka == phi)
    row = lax.broadcasted_iota(jnp.int32, (_K, _K), 0)
    col = lax.broadcasted_iota(jnp.int32, (_K, _K), 1)
    tri = (row <= col).astype(jnp.float32)
    stacked = jnp.concatenate(
        [gt.astype(jnp.float32), eq.astype(jnp.float32)], axis=0)
    cums = lax.dot_general(
        stacked, tri, (((1,), (0,)), ((), ())),
        precision=lax.Precision.HIGHEST)
    cum_gt = cums[:_B]
    cum_eq = cums[_B:]
    r = FG - cnt_gt
    sel_eq = eq & (cum_eq <= r + 0.5)
    sel = gt | sel_eq
    cum_sel = cum_gt + jnp.minimum(cum_eq, r)
    rank_ref[...] = jnp.where(sel, cum_sel, 0.0)


def _compact_kernel(rank_ref, idx_ref):
    rank = rank_ref[0]
    q = lax.broadcasted_iota(jnp.int32, (2 * FG, _K), 0)
    qmod = q & 15
    target = ((q >> 4) * 8 + qmod + 1).astype(jnp.float32)
    valid = qmod < 8
    onehot = jnp.where(
        valid & (jnp.broadcast_to(rank, (2 * FG, _K)) == target), 1.0, 0.0)
    kvec = lax.broadcasted_iota(jnp.int32, (_K, 1), 0).astype(jnp.float32)
    kidx = lax.dot_general(onehot, kvec, (((1,), (0,)), ((), ())),
                           precision=lax.Precision.HIGHEST)
    idx_ref[0] = kidx.astype(jnp.int32)


def _make_gather():
    mesh = plsc.VectorSubcoreMesh(core_axis_name="c", subcore_axis_name="s")

    grp = 8
    ngrp = FG // grp

    @functools.partial(
        pl.kernel, mesh=mesh,
        out_type=jax.ShapeDtypeStruct((_B, 32, 32), jnp.float32),
        scratch_types=[
            pltpu.VMEM((_BPW, 2 * FG), jnp.int32),
            pltpu.VMEM((2, grp, 32, 32), jnp.float32),
            pltpu.VMEM((_BPW, 32, 32), jnp.float32),
            pltpu.SemaphoreType.DMA((2,)),
        ],
    )
    def gather(idx_hbm, hm_hbm, out_hbm, idx_s, buf, acc, sems):
        cid = lax.axis_index("c")
        sid = lax.axis_index("s")
        wid = sid * 2 + cid
        b0 = wid * _BPW
        pltpu.sync_copy(idx_hbm.at[pl.ds(b0, _BPW)], idx_s)
        for g in range(_BPW * _P // 16):
            p = g % (_P // 16)
            acc[g // (_P // 16), p // 2, pl.ds((p % 2) * 16, 16)] = (
                jnp.zeros((16,), jnp.float32))

        def issue(s, g, par):
            kv = idx_s[s, pl.ds(g * 16, 16)]
            for e in range(grp):
                pltpu.async_copy(hm_hbm.at[b0 + s, kv[e]],
                                 buf.at[par, e], sems.at[par])

        def drain_add(s, par):
            pltpu.make_async_copy(hm_hbm.at[0, pl.ds(0, grp)],
                                  buf.at[par], sems.at[par]).wait()

            def row_body(rr, c, s=s, par=par):
                for g in range(_P // 16):
                    acc[s, g // 2, pl.ds((g % 2) * 16, 16)] += (
                        buf[par, rr, g // 2, pl.ds((g % 2) * 16, 16)])
                return c

            lax.fori_loop(0, grp, row_body, 0)

        for s in range(_BPW):
            issue(s, 0, 0)

            def gp_body(gp, carry, s=s):
                g0 = gp * 2
                issue(s, g0 + 1, 1)
                drain_add(s, 0)

                @pl.when(gp < ngrp // 2 - 1)
                def _():
                    issue(s, g0 + 2, 0)

                drain_add(s, 1)
                return carry

            lax.fori_loop(0, ngrp // 2, gp_body, 0)
        pltpu.sync_copy(acc, out_hbm.at[pl.ds(b0, _BPW)])

    return gather


def _js_kernel(cs_ref, av_ref, out_ref):
    comb = cs_ref[...] * (1.0 / FG)
    cmax = jnp.max(comb, axis=(1, 2), keepdims=True)
    ce = jnp.exp(comb - cmax)
    att = ce / jnp.sum(ce, axis=(1, 2), keepdims=True)

    av = av_ref[...]
    vmax = jnp.max(av, axis=(1, 2), keepdims=True)
    ve = jnp.exp(av - vmax)
    avd = ve / jnp.sum(ve, axis=(1, 2), keepdims=True)

    lm = jnp.log((att + avd) * 0.5)
    tot = jnp.sum(att * (jnp.log(att) - lm) + avd * (jnp.log(avd) - lm))
    out_ref[...] = jnp.full((8, 128), tot, jnp.float32)


def kernel(att_feat, aud_feat, att_heatmaps, av_heatmaps):
    B, C, K = att_feat.shape
    P = att_heatmaps.shape[2] * att_heatmaps.shape[3]

    sim = pl.pallas_call(
        _sim_kernel,
        grid=(B,),
        in_specs=[
            pl.BlockSpec((1, C, K), lambda b: (b, 0, 0)),
            pl.BlockSpec((1, 1, C), lambda b: (b, 0, 0)),
        ],
        out_specs=pl.BlockSpec((1, 1, K), lambda b: (b, 0, 0)),
        out_shape=jax.ShapeDtypeStruct((B, 1, K), jnp.float32),
    )(att_feat, aud_feat.reshape(B, 1, C)).reshape(B, K)

    dis, rank = pl.pallas_call(
        _select_kernel,
        in_specs=[pl.BlockSpec((B, K), lambda: (0, 0))],
        out_specs=[
            pl.BlockSpec((8, 128), lambda: (0, 0)),
            pl.BlockSpec((B, K), lambda: (0, 0)),
        ],
        out_shape=[
            jax.ShapeDtypeStruct((8, 128), jnp.float32),
            jax.ShapeDtypeStruct((B, K), jnp.float32),
        ],
    )(sim)

    idx = pl.pallas_call(
        _compact_kernel,
        grid=(B,),
        in_specs=[pl.BlockSpec((1, 1, K), lambda b: (b, 0, 0))],
        out_specs=pl.BlockSpec((1, 2 * FG, 1), lambda b: (b, 0, 0)),
        out_shape=jax.ShapeDtypeStruct((B, 2 * FG, 1), jnp.int32),
    )(rank.reshape(B, 1, K))

    comb_sum = _make_gather()(idx.reshape(B, 2 * FG), att_heatmaps)

    div = pl.pallas_call(
        _js_kernel,
        in_specs=[
            pl.BlockSpec((B, 32, 32), lambda: (0, 0, 0)),
            pl.BlockSpec((B, 32, 32), lambda: (0, 0, 0)),
        ],
        out_specs=pl.BlockSpec((8, 128), lambda: (0, 0)),
        out_shape=jax.ShapeDtypeStruct((8, 128), jnp.float32),
    )(comb_sum, av_heatmaps.reshape(B, 32, 32))

    dis_loss = dis[0, 0].reshape(())
    div_loss = (div[0, 0] / (2.0 * B)).reshape(())
    return dis_loss, div_loss


# speedup vs baseline: 2.3491x; 2.3112x over previous
import jax
import jax.numpy as jnp
from jax.experimental import pallas as pl
from jax.experimental.pallas import tpu as pltpu

FG = 128
BG = 128
_B, _C, _K = 64, 512, 1024
_P = 1024
_I32_MIN = -2147483648
_M31 = 2147483647


def _sim_kernel(att_ref, aud_ref, sim_ref):
    a = att_ref[0]
    aud = aud_ref[0]
    dot = jnp.dot(aud, a, preferred_element_type=jnp.float32)
    nsq = jnp.sum(a * a, axis=0, keepdims=True)
    sim_ref[0] = dot / jnp.maximum(jnp.sqrt(nsq), 1e-12)


def _key(x_i32):
    return jnp.where(x_i32 < 0, x_i32 ^ jnp.int32(_M31), x_i32)


def _select_kernel(sim_ref, dis_ref, w_ref):
    sim = sim_ref[...]
    ka = _key(jax.lax.bitcast_convert_type(sim, jnp.int32))

    def body(j, P):
        bit = 31 - j
        phi, plo = P
        step = jnp.int32(1) << bit
        chi = phi + step
        clo = plo + step
        cnt_hi = jnp.sum((ka >= chi).astype(jnp.int32), axis=1, keepdims=True)
        cnt_lo = jnp.sum((ka >= clo).astype(jnp.int32), axis=1, keepdims=True)
        phi = jnp.where(cnt_hi >= FG, chi, phi)
        plo = jnp.where(cnt_lo >= _K - BG + 1, clo, plo)
        return (phi, plo)

    p0 = jnp.full((_B, 1), _I32_MIN, jnp.int32)
    phi, plo = jax.lax.fori_loop(0, 32, body, (p0, p0))

    thi_f = jax.lax.bitcast_convert_type(_key(phi), jnp.float32)
    tlo_f = jax.lax.bitcast_convert_type(_key(plo), jnp.float32)

    gt = ka > phi
    cnt_gt = jnp.sum(gt.astype(jnp.float32), axis=1, keepdims=True)
    sum_gt = jnp.sum(jnp.where(gt, sim, 0.0), axis=1, keepdims=True)
    pos = (sum_gt + thi_f * (FG - cnt_gt)) * (1.0 / FG)

    lt = ka < plo
    cnt_lt = jnp.sum(lt.astype(jnp.float32), axis=1, keepdims=True)
    sum_lt = jnp.sum(jnp.where(lt, sim, 0.0), axis=1, keepdims=True)
    hn = (sum_lt + tlo_f * (BG - cnt_lt)) * (1.0 / BG)

    m = jnp.maximum(pos, hn)
    logz = m + jnp.log(jnp.exp(pos - m) + jnp.exp(hn - m))
    dis = jnp.mean(logz - pos)
    dis_ref[...] = jnp.full((8, 128), dis, jnp.float32)

    eq = (ka == phi)
    row = jax.lax.broadcasted_iota(jnp.int32, (_K, _K), 0)
    col = jax.lax.broadcasted_iota(jnp.int32, (_K, _K), 1)
    tri = (row <= col).astype(jnp.float32)
    cum_eq = jax.lax.dot_general(
        eq.astype(jnp.float32), tri, (((1,), (0,)), ((), ())),
        precision=jax.lax.Precision.HIGHEST)
    r = FG - cnt_gt
    sel = gt | (eq & (cum_eq <= r + 0.5))
    w_ref[...] = sel.astype(jnp.float32) * (1.0 / FG)


def _combine_kernel(w_ref, hm_ref, av_ref, acc_ref):
    b = pl.program_id(0)
    w = w_ref[0]
    h = hm_ref[0]
    comb = jnp.dot(w, h, preferred_element_type=jnp.float32,
                   precision=jax.lax.Precision.HIGHEST)

    cmax = jnp.max(comb, axis=1, keepdims=True)
    ce = jnp.exp(comb - cmax)
    att = ce / jnp.sum(ce, axis=1, keepdims=True)

    av = av_ref[0]
    vmax = jnp.max(av, axis=1, keepdims=True)
    ve = jnp.exp(av - vmax)
    avd = ve / jnp.sum(ve, axis=1, keepdims=True)

    lm = jnp.log((att + avd) * 0.5)
    term = (jnp.sum(att * (jnp.log(att) - lm)) +
            jnp.sum(avd * (jnp.log(avd) - lm)))

    @pl.when(b == 0)
    def _():
        acc_ref[...] = jnp.zeros_like(acc_ref)

    acc_ref[...] += jnp.full((8, 128), term, jnp.float32)


def kernel(att_feat, aud_feat, att_heatmaps, av_heatmaps):
    B, C, K = att_feat.shape
    P = att_heatmaps.shape[2] * att_heatmaps.shape[3]

    sim = pl.pallas_call(
        _sim_kernel,
        grid=(B,),
        in_specs=[
            pl.BlockSpec((1, C, K), lambda b: (b, 0, 0)),
            pl.BlockSpec((1, 1, C), lambda b: (b, 0, 0)),
        ],
        out_specs=pl.BlockSpec((1, 1, K), lambda b: (b, 0, 0)),
        out_shape=jax.ShapeDtypeStruct((B, 1, K), jnp.float32),
    )(att_feat, aud_feat.reshape(B, 1, C)).reshape(B, K)

    dis, w = pl.pallas_call(
        _select_kernel,
        in_specs=[pl.BlockSpec((B, K), lambda: (0, 0))],
        out_specs=[
            pl.BlockSpec((8, 128), lambda: (0, 0)),
            pl.BlockSpec((B, K), lambda: (0, 0)),
        ],
        out_shape=[
            jax.ShapeDtypeStruct((8, 128), jnp.float32),
            jax.ShapeDtypeStruct((B, K), jnp.float32),
        ],
    )(sim)

    acc = pl.pallas_call(
        _combine_kernel,
        grid=(B,),
        in_specs=[
            pl.BlockSpec((1, 1, K), lambda b: (b, 0, 0)),
            pl.BlockSpec((1, K, P), lambda b: (b, 0, 0)),
            pl.BlockSpec((1, 1, P), lambda b: (b, 0, 0)),
        ],
        out_specs=pl.BlockSpec((8, 128), lambda b: (0, 0)),
        out_shape=jax.ShapeDtypeStruct((8, 128), jnp.float32),
    )(w.reshape(B, 1, K), att_heatmaps.reshape(B, K, P),
      av_heatmaps.reshape(B, 1, P))

    dis_loss = dis[0, 0].reshape(())
    div_loss = (acc[0, 0] / (2.0 * B)).reshape(())
    return dis_loss, div_loss
